# Initial kernel scaffold; baseline (speedup 1.0000x reference)
import jax
import jax.numpy as jnp
from jax.experimental import pallas as pl


def kernel(x, edge_index, W1, b1, W2, b2):
    raise NotImplementedError("write your pallas kernel here")


# speedup vs baseline: 9.1971x; 9.1971x over previous
import functools

import jax
import jax.numpy as jnp
from jax import lax
from jax.experimental import pallas as pl
from jax.experimental.pallas import tpu as pltpu
from jax.experimental.pallas import tpu_sc as plsc

N_NODES = 10000
F = 128
HID = 256
K = 10
ALPHA = 0.1

N_PAD = 10240
NW = 32
EDGES_PER_TILE = 10000
EB = 128
NB = 80
NB = (EDGES_PER_TILE + EB - 1) // EB
EDGES_PAD_PER_TILE = NB * EB
ROWS_PER_TILE = N_PAD // 16


def _sc_scatter_body(g_hbm, src_hbm, dst_hbm, zeros_hbm, out_hbm,
                     accum, idx_s, idx_d, rows, sem_g, sem_s):
    c = lax.axis_index("c")
    s = lax.axis_index("s")
    wid = c * 16 + s

    pltpu.sync_copy(zeros_hbm.at[pl.ds(s * ROWS_PER_TILE, ROWS_PER_TILE)],
                    accum.at[pl.ds(s * ROWS_PER_TILE, ROWS_PER_TILE)])
    plsc.subcore_barrier()

    base = wid * NB

    def body(b, carry):
        pltpu.sync_copy(src_hbm.at[base + b], idx_s)
        pltpu.sync_copy(dst_hbm.at[base + b], idx_d)
        pltpu.async_copy(g_hbm.at[idx_s], rows, sem_g).wait()
        pltpu.async_copy(rows, accum.at[idx_d], sem_s, add=True).wait()
        return carry

    lax.fori_loop(0, NB, body, 0)

    plsc.subcore_barrier()
    pltpu.sync_copy(accum.at[pl.ds(s * ROWS_PER_TILE, ROWS_PER_TILE)],
                    out_hbm.at[c, pl.ds(s * ROWS_PER_TILE, ROWS_PER_TILE)])


_sc_scatter = pl.kernel(
    _sc_scatter_body,
    mesh=plsc.VectorSubcoreMesh(core_axis_name="c", subcore_axis_name="s"),
    out_type=jax.ShapeDtypeStruct((2, N_PAD, F), jnp.float32),
    scratch_types=[
        pltpu.VMEM_SHARED((N_PAD, F), jnp.float32),
        pltpu.VMEM((EB,), jnp.int32),
        pltpu.VMEM((EB,), jnp.int32),
        pltpu.VMEM((EB, F), jnp.float32),
        pltpu.SemaphoreType.DMA,
        pltpu.SemaphoreType.DMA,
    ],
)


def _mlp_body(x_ref, w1_ref, b1_ref, w2_ref, b2_ref, da_ref, db_ref,
              h0_ref, g0_ref, q_ref, dis_ref):
    deg = da_ref[:, 0:1] + db_ref[:, 0:1] + 1.0
    dis = lax.rsqrt(deg)
    h = jnp.maximum(
        jnp.dot(x_ref[...], w1_ref[...], preferred_element_type=jnp.float32)
        + b1_ref[...], 0.0)
    h = jnp.dot(h, w2_ref[...], preferred_element_type=jnp.float32) + b2_ref[...]
    h0_ref[...] = h
    g0_ref[...] = dis * h
    q_ref[...] = 1.0 / deg
    dis_ref[...] = dis


def _update_body(sa_ref, sb_ref, g_ref, base_ref, vec_ref, out_ref):
    out_ref[...] = ((1.0 - ALPHA) * vec_ref[...]
                    * (sa_ref[...] + sb_ref[...] + g_ref[...])
                    + ALPHA * base_ref[...])


_BLK = 512
_GRID = N_PAD // _BLK


def _row_spec(width):
    return pl.BlockSpec((_BLK, width), lambda i: (i, 0))


_mlp_call = pl.pallas_call(
    _mlp_body,
    grid=(_GRID,),
    in_specs=[
        _row_spec(F),
        pl.BlockSpec((F, HID), lambda i: (0, 0)),
        pl.BlockSpec((1, HID), lambda i: (0, 0)),
        pl.BlockSpec((HID, F), lambda i: (0, 0)),
        pl.BlockSpec((1, F), lambda i: (0, 0)),
        _row_spec(F),
        _row_spec(F),
    ],
    out_specs=[_row_spec(F), _row_spec(F), _row_spec(1), _row_spec(1)],
    out_shape=[
        jax.ShapeDtypeStruct((N_PAD, F), jnp.float32),
        jax.ShapeDtypeStruct((N_PAD, F), jnp.float32),
        jax.ShapeDtypeStruct((N_PAD, 1), jnp.float32),
        jax.ShapeDtypeStruct((N_PAD, 1), jnp.float32),
    ],
)

_update_call = pl.pallas_call(
    _update_body,
    grid=(_GRID,),
    in_specs=[_row_spec(F), _row_spec(F), _row_spec(F), _row_spec(F),
              _row_spec(1)],
    out_specs=_row_spec(F),
    out_shape=jax.ShapeDtypeStruct((N_PAD, F), jnp.float32),
)


def kernel(x, edge_index, W1, b1, W2, b2):
    src = edge_index[0].astype(jnp.int32).reshape(NW, EDGES_PER_TILE)
    dst = edge_index[1].astype(jnp.int32).reshape(NW, EDGES_PER_TILE)

    n_extra = EDGES_PAD_PER_TILE - EDGES_PER_TILE
    w_ids = jnp.arange(NW, dtype=jnp.int32)[:, None]
    j_ids = jnp.arange(n_extra, dtype=jnp.int32)[None, :]
    pad_src = (w_ids * 37 + j_ids * 89) % N_NODES
    pad_dst = N_NODES + (w_ids * 13 + j_ids * 7) % (N_PAD - N_NODES)
    src_t = jnp.concatenate([src, pad_src], axis=1).reshape(NW * NB, EB)
    dst_t = jnp.concatenate([dst, pad_dst], axis=1).reshape(NW * NB, EB)

    zeros = jnp.zeros((N_PAD, F), jnp.float32)
    ones = jnp.ones((N_PAD, F), jnp.float32)

    deg_parts = _sc_scatter(ones, src_t, dst_t, zeros)

    x_pad = jnp.zeros((N_PAD, F), x.dtype).at[:N_NODES].set(x)
    h0, g0, q, dis = _mlp_call(x_pad, W1, b1.reshape(1, HID), W2,
                               b2.reshape(1, F), deg_parts[0], deg_parts[1])

    g = g0
    for k in range(K):
        s_parts = _sc_scatter(g, src_t, dst_t, zeros)
        if k < K - 1:
            g = _update_call(s_parts[0], s_parts[1], g, g0, q)
        else:
            h = _update_call(s_parts[0], s_parts[1], g, h0, dis)
    return h[:N_NODES]


# speedup vs baseline: 15.8244x; 1.7206x over previous
import functools

import jax
import jax.numpy as jnp
from jax import lax
from jax.experimental import pallas as pl
from jax.experimental.pallas import tpu as pltpu
from jax.experimental.pallas import tpu_sc as plsc

N_NODES = 10000
F = 128
HID = 256
K = 10
ALPHA = 0.1

N_PAD = 10240
NW = 32
EDGES_PER_TILE = 10000
EB = 128
NB = 80
EDGES_PAD_PER_TILE = NB * EB
ROWS_PER_TILE = N_PAD // 16


def _sc_scatter_body(g_hbm, src_hbm, dst_hbm, zeros_hbm, out_hbm,
                     accum, isrc0, isrc1, idx_d, rows0, rows1,
                     sem_g0, sem_g1, sem_s0, sem_s1):
    c = lax.axis_index("c")
    s = lax.axis_index("s")
    wid = c * 16 + s

    zcp = pltpu.async_copy(
        zeros_hbm.at[pl.ds(s * ROWS_PER_TILE, ROWS_PER_TILE)],
        accum.at[pl.ds(s * ROWS_PER_TILE, ROWS_PER_TILE)], sem_s0)
    pltpu.sync_copy(dst_hbm.at[pl.ds(wid * NB, NB)], idx_d)
    pltpu.sync_copy(src_hbm.at[wid * NB], isrc0)
    pltpu.sync_copy(src_hbm.at[wid * NB + 1], isrc1)
    zcp.wait()
    plsc.subcore_barrier()

    def gather(ibuf, buf, sem):
        return pltpu.async_copy(g_hbm.at[ibuf], buf, sem)

    def scatter(b, buf, sem):
        return pltpu.async_copy(buf, accum.at[idx_d.at[b]], sem, add=True)

    gather(isrc0, rows0, sem_g0)

    def body(i, carry):
        b0 = 2 * i
        b1 = 2 * i + 1
        b2 = jnp.minimum(2 * i + 2, NB - 1)
        b3 = jnp.minimum(2 * i + 3, NB - 1)
        pltpu.make_async_copy(g_hbm.at[isrc0], rows0, sem_g0).wait()
        gather(isrc1, rows1, sem_g1)
        scatter(b0, rows0, sem_s0)
        pltpu.sync_copy(src_hbm.at[wid * NB + b2], isrc0)
        pltpu.make_async_copy(g_hbm.at[isrc1], rows1, sem_g1).wait()
        pltpu.make_async_copy(rows0, accum.at[idx_d.at[b0]], sem_s0).wait()
        gather(isrc0, rows0, sem_g0)
        scatter(b1, rows1, sem_s1)
        pltpu.sync_copy(src_hbm.at[wid * NB + b3], isrc1)
        pltpu.make_async_copy(rows1, accum.at[idx_d.at[b1]], sem_s1).wait()
        return carry

    lax.fori_loop(0, NB // 2, body, 0)
    pltpu.make_async_copy(g_hbm.at[isrc0], rows0, sem_g0).wait()

    plsc.subcore_barrier()
    pltpu.sync_copy(accum.at[pl.ds(s * ROWS_PER_TILE, ROWS_PER_TILE)],
                    out_hbm.at[c, pl.ds(s * ROWS_PER_TILE, ROWS_PER_TILE)])


_sc_scatter = pl.kernel(
    _sc_scatter_body,
    mesh=plsc.VectorSubcoreMesh(core_axis_name="c", subcore_axis_name="s"),
    out_type=jax.ShapeDtypeStruct((2, N_PAD, F), jnp.float32),
    scratch_types=[
        pltpu.VMEM_SHARED((N_PAD, F), jnp.float32),
        pltpu.VMEM((EB,), jnp.int32),
        pltpu.VMEM((EB,), jnp.int32),
        pltpu.VMEM((NB, EB), jnp.int32),
        pltpu.VMEM((EB, F), jnp.float32),
        pltpu.VMEM((EB, F), jnp.float32),
        pltpu.SemaphoreType.DMA,
        pltpu.SemaphoreType.DMA,
        pltpu.SemaphoreType.DMA,
        pltpu.SemaphoreType.DMA,
    ],
)


def _mlp_body(x_ref, w1_ref, b1_ref, w2_ref, b2_ref, da_ref, db_ref,
              h0_ref, g0_ref, q_ref, dis_ref):
    deg = da_ref[:, 0:1] + db_ref[:, 0:1] + 1.0
    dis = lax.rsqrt(deg)
    h = jnp.maximum(
        jnp.dot(x_ref[...], w1_ref[...], preferred_element_type=jnp.float32)
        + b1_ref[...], 0.0)
    h = jnp.dot(h, w2_ref[...], preferred_element_type=jnp.float32) + b2_ref[...]
    h0_ref[...] = h
    g0_ref[...] = dis * h
    q_ref[...] = 1.0 / deg
    dis_ref[...] = dis


def _update_body(sa_ref, sb_ref, g_ref, base_ref, vec_ref, out_ref):
    out_ref[...] = ((1.0 - ALPHA) * vec_ref[...]
                    * (sa_ref[...] + sb_ref[...] + g_ref[...])
                    + ALPHA * base_ref[...])


_BLK = 512
_GRID = N_PAD // _BLK


def _row_spec(width):
    return pl.BlockSpec((_BLK, width), lambda i: (i, 0))


_mlp_call = pl.pallas_call(
    _mlp_body,
    grid=(_GRID,),
    in_specs=[
        _row_spec(F),
        pl.BlockSpec((F, HID), lambda i: (0, 0)),
        pl.BlockSpec((1, HID), lambda i: (0, 0)),
        pl.BlockSpec((HID, F), lambda i: (0, 0)),
        pl.BlockSpec((1, F), lambda i: (0, 0)),
        _row_spec(F),
        _row_spec(F),
    ],
    out_specs=[_row_spec(F), _row_spec(F), _row_spec(1), _row_spec(1)],
    out_shape=[
        jax.ShapeDtypeStruct((N_PAD, F), jnp.float32),
        jax.ShapeDtypeStruct((N_PAD, F), jnp.float32),
        jax.ShapeDtypeStruct((N_PAD, 1), jnp.float32),
        jax.ShapeDtypeStruct((N_PAD, 1), jnp.float32),
    ],
)

_update_call = pl.pallas_call(
    _update_body,
    grid=(_GRID,),
    in_specs=[_row_spec(F), _row_spec(F), _row_spec(F), _row_spec(F),
              _row_spec(1)],
    out_specs=_row_spec(F),
    out_shape=jax.ShapeDtypeStruct((N_PAD, F), jnp.float32),
)


def kernel(x, edge_index, W1, b1, W2, b2):
    src = edge_index[0].astype(jnp.int32).reshape(NW, EDGES_PER_TILE)
    dst = edge_index[1].astype(jnp.int32).reshape(NW, EDGES_PER_TILE)

    n_extra = EDGES_PAD_PER_TILE - EDGES_PER_TILE
    w_ids = jnp.arange(NW, dtype=jnp.int32)[:, None]
    j_ids = jnp.arange(n_extra, dtype=jnp.int32)[None, :]
    pad_src = (w_ids * 37 + j_ids * 89) % N_NODES
    pad_dst = N_NODES + (w_ids * 13 + j_ids * 7) % (N_PAD - N_NODES)
    src_t = jnp.concatenate([src, pad_src], axis=1).reshape(NW * NB, EB)
    dst_t = jnp.concatenate([dst, pad_dst], axis=1).reshape(NW * NB, EB)

    zeros = jnp.zeros((N_PAD, F), jnp.float32)
    ones = jnp.ones((N_PAD, F), jnp.float32)

    deg_parts = _sc_scatter(ones, src_t, dst_t, zeros)

    x_pad = jnp.zeros((N_PAD, F), x.dtype).at[:N_NODES].set(x)
    h0, g0, q, dis = _mlp_call(x_pad, W1, b1.reshape(1, HID), W2,
                               b2.reshape(1, F), deg_parts[0], deg_parts[1])

    g = g0
    for k in range(K):
        s_parts = _sc_scatter(g, src_t, dst_t, zeros)
        if k < K - 1:
            g = _update_call(s_parts[0], s_parts[1], g, g0, q)
        else:
            h = _update_call(s_parts[0], s_parts[1], g, h0, dis)
    return h[:N_NODES]
